# baseline (speedup 1.0000x reference)
import jax
import jax.numpy as jnp
from jax import lax
from jax.experimental import pallas as pl
from jax.experimental.pallas import tpu as pltpu
from jax.experimental.pallas import tpu_sc as plsc

_NUM_EXPERTS = 16
_L = 16
_NS = 16
_NUM_LAYERS = 24
_TOKENS = 4 * 4096
_TOK_PER_TILE = _TOKENS // _NS
_CHUNKS = _TOK_PER_TILE // _L
_BANKS = 8

_TOTAL_CAPITAL = 10000.0
_MIN_CAP = _TOTAL_CAPITAL * 0.05 / _NUM_EXPERTS
_TAX_THRESHOLD = 2.0
_TAX_RATE = 0.1


_OFF_LC = _TOKENS
_OFF_W0 = 2 * _TOKENS
_OFF_W1 = 3 * _TOKENS
_OFF_LI = 4 * _TOKENS
_OFF_CAPS = _OFF_LI + _L
_OFF_BASE = _OFF_CAPS + _NUM_LAYERS * _NUM_EXPERTS
_BIG = _OFF_BASE + _NUM_LAYERS


def _sc_body(big_h,
             caps_out, base_out,
             loss_v, lc_v, w0_v, w1_v,
             c0, c1, c2, c3, c4, c5, c6, c7,
             s0, s1, s2, s3, s4, s5, s6, s7, part_v,
             li_v, base24_v, caps24_v, shared, all_v, sem, sem2):
    cnt_bs = (c0, c1, c2, c3, c4, c5, c6, c7)
    s_bs = (s0, s1, s2, s3, s4, s5, s6, s7)
    sid = lax.axis_index("s")
    off = sid * _TOK_PER_TILE
    d1 = pltpu.async_copy(big_h.at[pl.ds(off, _TOK_PER_TILE)], loss_v, sem)
    d2 = pltpu.async_copy(big_h.at[pl.ds(_OFF_LC + off, _TOK_PER_TILE)],
                          lc_v, sem)
    d3 = pltpu.async_copy(big_h.at[pl.ds(_OFF_W0 + off, _TOK_PER_TILE)],
                          w0_v, sem)
    d4 = pltpu.async_copy(big_h.at[pl.ds(_OFF_W1 + off, _TOK_PER_TILE)],
                          w1_v, sem)

    @pl.when(sid == 0)
    def _prefetch():
        pltpu.async_copy(big_h.at[pl.ds(_OFF_LI, _L)], li_v, sem2)
        pltpu.async_copy(
            big_h.at[pl.ds(_OFF_CAPS, _NUM_LAYERS * _NUM_EXPERTS)],
            caps24_v, sem2)
        pltpu.async_copy(big_h.at[pl.ds(_OFF_BASE, _NUM_LAYERS)],
                         base24_v, sem2)

    zeros = jnp.zeros((_L,), jnp.float32)
    ones = jnp.ones((_L,), jnp.float32)
    lanes = lax.iota(jnp.int32, _L)
    for r in cnt_bs + s_bs:
        r[...] = zeros

    d1.wait()
    d2.wait()
    d3.wait()
    d4.wait()

    loss_accs = [zeros] * 4
    for i in range(_CHUNKS):
        sl = pl.ds(i * _L, _L)
        lo = loss_v[sl]
        lc = lc_v[sl]
        a = plsc.bitcast(w0_v[sl], jnp.int32)
        b = plsc.bitcast(w1_v[sl], jnp.int32)
        m = b != a
        bk0 = (2 * i) % _BANKS
        bk1 = (2 * i + 1) % _BANKS
        plsc.addupdate_scatter(cnt_bs[bk0], [a], ones)
        plsc.addupdate_scatter(s_bs[bk0], [a], lc)
        plsc.addupdate_scatter(cnt_bs[bk1], [b], ones, mask=m)
        plsc.addupdate_scatter(s_bs[bk1], [b], lc, mask=m)
        loss_accs[i % 4] = loss_accs[i % 4] + lo

    cnt_p = zeros
    s_p = zeros
    for bnk in range(_BANKS):
        cnt_p = cnt_p + cnt_bs[bnk][...]
        s_p = s_p + s_bs[bnk][...]
    loss_acc = (loss_accs[0] + loss_accs[1]) + (loss_accs[2] + loss_accs[3])

    part_v[pl.ds(0, _L)] = cnt_p
    part_v[pl.ds(_L, _L)] = s_p
    part_v[pl.ds(2 * _L, _L)] = loss_acc
    pltpu.sync_copy(part_v, shared.at[pl.ds(sid * 3 * _L, 3 * _L)])
    plsc.subcore_barrier()

    @pl.when(sid == 0)
    def _finalize():
        pltpu.sync_copy(shared, all_v)
        pltpu.make_async_copy(big_h.at[pl.ds(_OFF_LI, _L)], li_v,
                              sem2).wait()
        pltpu.make_async_copy(
            big_h.at[pl.ds(_OFF_CAPS, _NUM_LAYERS * _NUM_EXPERTS)],
            caps24_v, sem2).wait()
        pltpu.make_async_copy(big_h.at[pl.ds(_OFF_BASE, _NUM_LAYERS)],
                              base24_v, sem2).wait()
        li_vec = plsc.bitcast(li_v[...], jnp.int32)
        cnt = all_v[pl.ds(0, _L)]
        s = all_v[pl.ds(_L, _L)]
        lsum = all_v[pl.ds(2 * _L, _L)]
        for i in range(1, _NS):
            cnt = cnt + all_v[pl.ds(i * 3 * _L, _L)]
            s = s + all_v[pl.ds((i * 3 + 1) * _L, _L)]
            lsum = lsum + all_v[pl.ds((i * 3 + 2) * _L, _L)]
        avg_loss = jnp.sum(lsum) * (1.0 / _TOKENS)
        base_splat = plsc.load_gather(base24_v, [li_vec])
        new_base = 0.99 * base_splat + 0.01 * avg_loss
        row_idx = li_vec * _NUM_EXPERTS + lanes
        crow = plsc.load_gather(caps24_v, [row_idx])
        caps = crow + new_base * cnt - s
        thr = jnp.sum(caps) * (_TAX_THRESHOLD / _NUM_EXPERTS)
        caps = jnp.where(caps > thr, caps - (caps - thr) * _TAX_RATE, caps)
        caps = jnp.maximum(caps, _MIN_CAP)
        total = jnp.sum(caps)
        scale = jnp.where(total > _TOTAL_CAPITAL * 1.5, 0.95, 1.0)
        shift = jnp.where(total < _TOTAL_CAPITAL * 0.5,
                          _TOTAL_CAPITAL * 0.01, 0.0)
        caps = caps * scale + shift
        plsc.store_scatter(caps24_v, [row_idx], caps)
        plsc.store_scatter(base24_v, [li_vec], new_base, mask=lanes == 0)
        do1 = pltpu.async_copy(caps24_v, caps_out, sem2)
        do2 = pltpu.async_copy(base24_v, base_out, sem2)
        do1.wait()
        do2.wait()


_mesh = plsc.VectorSubcoreMesh(
    core_axis_name="c", subcore_axis_name="s", num_cores=1, num_subcores=_NS)

_sc_call = pl.kernel(
    _sc_body,
    out_type=(
        jax.ShapeDtypeStruct((_NUM_LAYERS * _NUM_EXPERTS,), jnp.float32),
        jax.ShapeDtypeStruct((_NUM_LAYERS,), jnp.float32),
    ),
    mesh=_mesh,
    scratch_types=[
        pltpu.VMEM((_TOK_PER_TILE,), jnp.float32),
        pltpu.VMEM((_TOK_PER_TILE,), jnp.float32),
        pltpu.VMEM((_TOK_PER_TILE,), jnp.float32),
        pltpu.VMEM((_TOK_PER_TILE,), jnp.float32),
        *([pltpu.VMEM((_L,), jnp.float32)] * (2 * _BANKS)),
        pltpu.VMEM((3 * _L,), jnp.float32),
        pltpu.VMEM((_L,), jnp.float32),
        pltpu.VMEM((_NUM_LAYERS,), jnp.float32),
        pltpu.VMEM((_NUM_LAYERS * _NUM_EXPERTS,), jnp.float32),
        pltpu.VMEM_SHARED((_NS * 3 * _L,), jnp.float32),
        pltpu.VMEM((_NS * 3 * _L,), jnp.float32),
        pltpu.SemaphoreType.DMA,
        pltpu.SemaphoreType.DMA,
    ],
    compiler_params=pltpu.CompilerParams(needs_layout_passes=False),
    name="capital_manager_sc",
)


def kernel(capitals, baseline_losses, token_losses, costs, winners, layer_idx):
    w = winners.reshape(_TOKENS, 2)
    big = jnp.concatenate([
        token_losses.reshape(_TOKENS),
        (token_losses + costs).reshape(_TOKENS),
        lax.bitcast_convert_type(w[:, 0], jnp.float32),
        lax.bitcast_convert_type(w[:, 1], jnp.float32),
        lax.bitcast_convert_type(
            jnp.full((_L,), layer_idx, dtype=jnp.int32), jnp.float32),
        capitals.reshape(_NUM_LAYERS * _NUM_EXPERTS),
        baseline_losses,
    ])
    caps_flat, new_baselines = _sc_call(big)
    return caps_flat.reshape(_NUM_LAYERS, _NUM_EXPERTS), new_baselines


# speedup vs baseline: 1.1169x; 1.1169x over previous
import jax
import jax.numpy as jnp
from jax import lax
from jax.experimental import pallas as pl
from jax.experimental.pallas import tpu as pltpu
from jax.experimental.pallas import tpu_sc as plsc

_NUM_EXPERTS = 16
_L = 16
_NS = 16
_NUM_LAYERS = 24
_TOKENS = 4 * 4096
_TOK_PER_TILE = _TOKENS // _NS
_CHUNKS = _TOK_PER_TILE // _L
_BANKS = 8

_TOTAL_CAPITAL = 10000.0
_MIN_CAP = _TOTAL_CAPITAL * 0.05 / _NUM_EXPERTS
_TAX_THRESHOLD = 2.0
_TAX_RATE = 0.1


def _sc_body(ll_h, w0li_h, w1_h, caps_h, base_h,
             caps_out, base_out,
             loss_v, lc_v, w0_v, w1_v,
             c0, c1, c2, c3, c4, c5, c6, c7,
             s0, s1, s2, s3, s4, s5, s6, s7, part_v,
             li_v, base24_v, caps24_v, shared, all_v, sem, sem2):
    cnt_bs = (c0, c1, c2, c3, c4, c5, c6, c7)
    s_bs = (s0, s1, s2, s3, s4, s5, s6, s7)
    sid = lax.axis_index("s")
    off = sid * _TOK_PER_TILE
    d1 = pltpu.async_copy(ll_h.at[pl.ds(off, _TOK_PER_TILE)], loss_v, sem)
    d2 = pltpu.async_copy(ll_h.at[pl.ds(_TOKENS + off, _TOK_PER_TILE)],
                          lc_v, sem)
    d3 = pltpu.async_copy(w0li_h.at[pl.ds(off, _TOK_PER_TILE)], w0_v, sem)
    d4 = pltpu.async_copy(w1_h.at[pl.ds(off, _TOK_PER_TILE)], w1_v, sem)

    @pl.when(sid == 0)
    def _prefetch():
        pltpu.async_copy(w0li_h.at[pl.ds(_TOKENS, _L)], li_v, sem2)
        pltpu.async_copy(caps_h, caps24_v, sem2)
        pltpu.async_copy(base_h, base24_v, sem2)

    zeros = jnp.zeros((_L,), jnp.float32)
    ones = jnp.ones((_L,), jnp.float32)
    lanes = lax.iota(jnp.int32, _L)
    for r in cnt_bs + s_bs:
        r[...] = zeros

    d1.wait()
    d2.wait()
    d3.wait()
    d4.wait()

    loss_accs = [zeros] * 4
    for i in range(_CHUNKS):
        sl = pl.ds(i * _L, _L)
        lo = loss_v[sl]
        lc = lc_v[sl]
        a = w0_v[sl]
        b = w1_v[sl]
        m = b != a
        bk0 = (2 * i) % _BANKS
        bk1 = (2 * i + 1) % _BANKS
        plsc.addupdate_scatter(cnt_bs[bk0], [a], ones)
        plsc.addupdate_scatter(s_bs[bk0], [a], lc)
        plsc.addupdate_scatter(cnt_bs[bk1], [b], ones, mask=m)
        plsc.addupdate_scatter(s_bs[bk1], [b], lc, mask=m)
        loss_accs[i % 4] = loss_accs[i % 4] + lo

    cnt_p = zeros
    s_p = zeros
    for bnk in range(_BANKS):
        cnt_p = cnt_p + cnt_bs[bnk][...]
        s_p = s_p + s_bs[bnk][...]
    loss_acc = (loss_accs[0] + loss_accs[1]) + (loss_accs[2] + loss_accs[3])

    part_v[pl.ds(0, _L)] = cnt_p
    part_v[pl.ds(_L, _L)] = s_p
    part_v[pl.ds(2 * _L, _L)] = loss_acc
    pltpu.sync_copy(part_v, shared.at[pl.ds(sid * 3 * _L, 3 * _L)])
    plsc.subcore_barrier()

    @pl.when(sid == 0)
    def _finalize():
        pltpu.sync_copy(shared, all_v)
        pltpu.make_async_copy(w0li_h.at[pl.ds(_TOKENS, _L)], li_v,
                              sem2).wait()
        pltpu.make_async_copy(caps_h, caps24_v, sem2).wait()
        pltpu.make_async_copy(base_h, base24_v, sem2).wait()
        li_vec = li_v[...]
        cnt = all_v[pl.ds(0, _L)]
        s = all_v[pl.ds(_L, _L)]
        lsum = all_v[pl.ds(2 * _L, _L)]
        for i in range(1, _NS):
            cnt = cnt + all_v[pl.ds(i * 3 * _L, _L)]
            s = s + all_v[pl.ds((i * 3 + 1) * _L, _L)]
            lsum = lsum + all_v[pl.ds((i * 3 + 2) * _L, _L)]
        avg_loss = jnp.sum(lsum) * (1.0 / _TOKENS)
        base_splat = plsc.load_gather(base24_v, [li_vec])
        new_base = 0.99 * base_splat + 0.01 * avg_loss
        crow = plsc.load_gather(caps24_v, [li_vec, lanes])
        caps = crow + new_base * cnt - s
        thr = jnp.sum(caps) * (_TAX_THRESHOLD / _NUM_EXPERTS)
        caps = jnp.where(caps > thr, caps - (caps - thr) * _TAX_RATE, caps)
        caps = jnp.maximum(caps, _MIN_CAP)
        total = jnp.sum(caps)
        scale = jnp.where(total > _TOTAL_CAPITAL * 1.5, 0.95, 1.0)
        shift = jnp.where(total < _TOTAL_CAPITAL * 0.5,
                          _TOTAL_CAPITAL * 0.01, 0.0)
        caps = caps * scale + shift
        plsc.store_scatter(caps24_v, [li_vec, lanes], caps)
        plsc.store_scatter(base24_v, [li_vec], new_base, mask=lanes == 0)
        do1 = pltpu.async_copy(caps24_v, caps_out, sem2)
        do2 = pltpu.async_copy(base24_v, base_out, sem2)
        do1.wait()
        do2.wait()


_mesh = plsc.VectorSubcoreMesh(
    core_axis_name="c", subcore_axis_name="s", num_cores=1, num_subcores=_NS)

_sc_call = pl.kernel(
    _sc_body,
    out_type=(
        jax.ShapeDtypeStruct((_NUM_LAYERS, _NUM_EXPERTS), jnp.float32),
        jax.ShapeDtypeStruct((_NUM_LAYERS,), jnp.float32),
    ),
    mesh=_mesh,
    scratch_types=[
        pltpu.VMEM((_TOK_PER_TILE,), jnp.float32),
        pltpu.VMEM((_TOK_PER_TILE,), jnp.float32),
        pltpu.VMEM((_TOK_PER_TILE,), jnp.int32),
        pltpu.VMEM((_TOK_PER_TILE,), jnp.int32),
        *([pltpu.VMEM((_L,), jnp.float32)] * (2 * _BANKS)),
        pltpu.VMEM((3 * _L,), jnp.float32),
        pltpu.VMEM((_L,), jnp.int32),
        pltpu.VMEM((_NUM_LAYERS,), jnp.float32),
        pltpu.VMEM((_NUM_LAYERS, _NUM_EXPERTS), jnp.float32),
        pltpu.VMEM_SHARED((_NS * 3 * _L,), jnp.float32),
        pltpu.VMEM((_NS * 3 * _L,), jnp.float32),
        pltpu.SemaphoreType.DMA,
        pltpu.SemaphoreType.DMA,
    ],
    compiler_params=pltpu.CompilerParams(needs_layout_passes=False),
    name="capital_manager_sc",
)


def kernel(capitals, baseline_losses, token_losses, costs, winners, layer_idx):
    w = winners.reshape(_TOKENS, 2)
    ll = jnp.concatenate([
        token_losses.reshape(_TOKENS),
        (token_losses + costs).reshape(_TOKENS),
    ])
    w0li = jnp.concatenate([
        w[:, 0],
        jnp.full((_L,), layer_idx, dtype=jnp.int32),
    ])
    new_capitals, new_baselines = _sc_call(
        ll, w0li, w[:, 1], capitals, baseline_losses)
    return new_capitals, new_baselines


# speedup vs baseline: 1.1422x; 1.0226x over previous
import jax
import jax.numpy as jnp
from jax import lax
from jax.experimental import pallas as pl
from jax.experimental.pallas import tpu as pltpu
from jax.experimental.pallas import tpu_sc as plsc

_NUM_EXPERTS = 16
_L = 16
_NS = 16
_NUM_LAYERS = 24
_TOKENS = 4 * 4096
_TOK_PER_TILE = _TOKENS // _NS
_CHUNKS = _TOK_PER_TILE // _L
_BANKS = 8

_TOTAL_CAPITAL = 10000.0
_MIN_CAP = _TOTAL_CAPITAL * 0.05 / _NUM_EXPERTS
_TAX_THRESHOLD = 2.0
_TAX_RATE = 0.1


def _sc_body(ll_h, w0li_h, caps_h, base_h,
             caps_out, base_out,
             loss_v, lc_v, w0_v, w1_v,
             c0, c1, c2, c3, c4, c5, c6, c7,
             s0, s1, s2, s3, s4, s5, s6, s7, part_v,
             li_v, base24_v, caps24_v, shared, all_v, sem, sem2):
    cnt_bs = (c0, c1, c2, c3, c4, c5, c6, c7)
    s_bs = (s0, s1, s2, s3, s4, s5, s6, s7)
    sid = lax.axis_index("s")
    off = sid * _TOK_PER_TILE
    d1 = pltpu.async_copy(ll_h.at[pl.ds(off, _TOK_PER_TILE)], loss_v, sem)
    d2 = pltpu.async_copy(ll_h.at[pl.ds(_TOKENS + off, _TOK_PER_TILE)],
                          lc_v, sem)
    d3 = pltpu.async_copy(w0li_h.at[pl.ds(off, _TOK_PER_TILE)], w0_v, sem)
    d4 = pltpu.async_copy(ll_h.at[pl.ds(2 * _TOKENS + off, _TOK_PER_TILE)],
                          w1_v, sem)

    @pl.when(sid == 0)
    def _prefetch():
        pltpu.async_copy(w0li_h.at[pl.ds(_TOKENS, _L)], li_v, sem2)
        pltpu.async_copy(caps_h, caps24_v, sem2)
        pltpu.async_copy(base_h, base24_v, sem2)

    zeros = jnp.zeros((_L,), jnp.float32)
    ones = jnp.ones((_L,), jnp.float32)
    lanes = lax.iota(jnp.int32, _L)
    for r in cnt_bs + s_bs:
        r[...] = zeros

    d1.wait()
    d2.wait()
    d3.wait()
    d4.wait()

    loss_accs = [zeros] * 4
    for i in range(_CHUNKS):
        sl = pl.ds(i * _L, _L)
        lo = loss_v[sl]
        lc = lc_v[sl]
        a = w0_v[sl]
        b = plsc.bitcast(w1_v[sl], jnp.int32)
        m = b != a
        bk0 = (2 * i) % _BANKS
        bk1 = (2 * i + 1) % _BANKS
        plsc.addupdate_scatter(cnt_bs[bk0], [a], ones)
        plsc.addupdate_scatter(s_bs[bk0], [a], lc)
        plsc.addupdate_scatter(cnt_bs[bk1], [b], ones, mask=m)
        plsc.addupdate_scatter(s_bs[bk1], [b], lc, mask=m)
        loss_accs[i % 4] = loss_accs[i % 4] + lo

    cnt_p = zeros
    s_p = zeros
    for bnk in range(_BANKS):
        cnt_p = cnt_p + cnt_bs[bnk][...]
        s_p = s_p + s_bs[bnk][...]
    loss_acc = (loss_accs[0] + loss_accs[1]) + (loss_accs[2] + loss_accs[3])

    part_v[pl.ds(0, _L)] = cnt_p
    part_v[pl.ds(_L, _L)] = s_p
    part_v[pl.ds(2 * _L, _L)] = loss_acc
    pltpu.sync_copy(part_v, shared.at[pl.ds(sid * 3 * _L, 3 * _L)])
    plsc.subcore_barrier()

    @pl.when(sid == 0)
    def _finalize():
        pltpu.sync_copy(shared, all_v)
        pltpu.make_async_copy(w0li_h.at[pl.ds(_TOKENS, _L)], li_v,
                              sem2).wait()
        pltpu.make_async_copy(caps_h, caps24_v, sem2).wait()
        pltpu.make_async_copy(base_h, base24_v, sem2).wait()
        li_vec = li_v[...]
        cnt = all_v[pl.ds(0, _L)]
        s = all_v[pl.ds(_L, _L)]
        lsum = all_v[pl.ds(2 * _L, _L)]
        for i in range(1, _NS):
            cnt = cnt + all_v[pl.ds(i * 3 * _L, _L)]
            s = s + all_v[pl.ds((i * 3 + 1) * _L, _L)]
            lsum = lsum + all_v[pl.ds((i * 3 + 2) * _L, _L)]
        avg_loss = jnp.sum(lsum) * (1.0 / _TOKENS)
        base_splat = plsc.load_gather(base24_v, [li_vec])
        new_base = 0.99 * base_splat + 0.01 * avg_loss
        crow = plsc.load_gather(caps24_v, [li_vec, lanes])
        caps = crow + new_base * cnt - s
        thr = jnp.sum(caps) * (_TAX_THRESHOLD / _NUM_EXPERTS)
        caps = jnp.where(caps > thr, caps - (caps - thr) * _TAX_RATE, caps)
        caps = jnp.maximum(caps, _MIN_CAP)
        total = jnp.sum(caps)
        scale = jnp.where(total > _TOTAL_CAPITAL * 1.5, 0.95, 1.0)
        shift = jnp.where(total < _TOTAL_CAPITAL * 0.5,
                          _TOTAL_CAPITAL * 0.01, 0.0)
        caps = caps * scale + shift
        plsc.store_scatter(caps24_v, [li_vec, lanes], caps)
        plsc.store_scatter(base24_v, [li_vec], new_base, mask=lanes == 0)
        do1 = pltpu.async_copy(caps24_v, caps_out, sem2)
        do2 = pltpu.async_copy(base24_v, base_out, sem2)
        do1.wait()
        do2.wait()


_mesh = plsc.VectorSubcoreMesh(
    core_axis_name="c", subcore_axis_name="s", num_cores=1, num_subcores=_NS)

_sc_call = pl.kernel(
    _sc_body,
    out_type=(
        jax.ShapeDtypeStruct((_NUM_LAYERS, _NUM_EXPERTS), jnp.float32),
        jax.ShapeDtypeStruct((_NUM_LAYERS,), jnp.float32),
    ),
    mesh=_mesh,
    scratch_types=[
        pltpu.VMEM((_TOK_PER_TILE,), jnp.float32),
        pltpu.VMEM((_TOK_PER_TILE,), jnp.float32),
        pltpu.VMEM((_TOK_PER_TILE,), jnp.int32),
        pltpu.VMEM((_TOK_PER_TILE,), jnp.float32),
        *([pltpu.VMEM((_L,), jnp.float32)] * (2 * _BANKS)),
        pltpu.VMEM((3 * _L,), jnp.float32),
        pltpu.VMEM((_L,), jnp.int32),
        pltpu.VMEM((_NUM_LAYERS,), jnp.float32),
        pltpu.VMEM((_NUM_LAYERS, _NUM_EXPERTS), jnp.float32),
        pltpu.VMEM_SHARED((_NS * 3 * _L,), jnp.float32),
        pltpu.VMEM((_NS * 3 * _L,), jnp.float32),
        pltpu.SemaphoreType.DMA,
        pltpu.SemaphoreType.DMA,
    ],
    compiler_params=pltpu.CompilerParams(needs_layout_passes=False),
    name="capital_manager_sc",
)


def kernel(capitals, baseline_losses, token_losses, costs, winners, layer_idx):
    w = winners.reshape(_TOKENS, 2)
    ll = jnp.concatenate([
        token_losses.reshape(_TOKENS),
        (token_losses + costs).reshape(_TOKENS),
        lax.bitcast_convert_type(w[:, 1], jnp.float32),
    ])
    w0li = jnp.concatenate([
        w[:, 0],
        jnp.full((_L,), layer_idx, dtype=jnp.int32),
    ])
    new_capitals, new_baselines = _sc_call(
        ll, w0li, capitals, baseline_losses)
    return new_capitals, new_baselines


# speedup vs baseline: 1.1466x; 1.0039x over previous
import jax
import jax.numpy as jnp
from jax import lax
from jax.experimental import pallas as pl
from jax.experimental.pallas import tpu as pltpu
from jax.experimental.pallas import tpu_sc as plsc

_NUM_EXPERTS = 16
_L = 16
_NS = 16
_NUM_LAYERS = 24
_TOKENS = 4 * 4096
_TOK_PER_TILE = _TOKENS // _NS
_CHUNKS = _TOK_PER_TILE // _L
_BANKS = 8

_TOTAL_CAPITAL = 10000.0
_MIN_CAP = _TOTAL_CAPITAL * 0.05 / _NUM_EXPERTS
_TAX_THRESHOLD = 2.0
_TAX_RATE = 0.1


def _sc_body(ll_h, w0li_h, cb_h,
             caps_out, base_out,
             loss_v, lc_v, w0_v, w1_v,
             c0, c1, c2, c3, c4, c5, c6, c7,
             s0, s1, s2, s3, s4, s5, s6, s7, part_v,
             li_v, base24_v, caps24_v, shared, all_v, sem, sem2):
    cnt_bs = (c0, c1, c2, c3, c4, c5, c6, c7)
    s_bs = (s0, s1, s2, s3, s4, s5, s6, s7)
    sid = lax.axis_index("s")
    off = sid * _TOK_PER_TILE
    d1 = pltpu.async_copy(ll_h.at[pl.ds(off, _TOK_PER_TILE)], loss_v, sem)
    d2 = pltpu.async_copy(ll_h.at[pl.ds(_TOKENS + off, _TOK_PER_TILE)],
                          lc_v, sem)
    d3 = pltpu.async_copy(w0li_h.at[pl.ds(off, _TOK_PER_TILE)], w0_v, sem)
    d4 = pltpu.async_copy(ll_h.at[pl.ds(2 * _TOKENS + off, _TOK_PER_TILE)],
                          w1_v, sem)

    @pl.when(sid == 0)
    def _prefetch():
        pltpu.async_copy(w0li_h.at[pl.ds(_TOKENS, _L)], li_v, sem2)
        pltpu.async_copy(cb_h.at[pl.ds(0, _NUM_LAYERS * _NUM_EXPERTS)],
                         caps24_v, sem2)
        pltpu.async_copy(
            cb_h.at[pl.ds(_NUM_LAYERS * _NUM_EXPERTS, _NUM_LAYERS)],
            base24_v, sem2)

    zeros = jnp.zeros((_L,), jnp.float32)
    ones = jnp.ones((_L,), jnp.float32)
    lanes = lax.iota(jnp.int32, _L)
    for r in cnt_bs + s_bs:
        r[...] = zeros

    d1.wait()
    d2.wait()
    d3.wait()
    d4.wait()

    loss_accs = [zeros] * 4
    for i in range(_CHUNKS):
        sl = pl.ds(i * _L, _L)
        lo = loss_v[sl]
        lc = lc_v[sl]
        a = w0_v[sl]
        b = plsc.bitcast(w1_v[sl], jnp.int32)
        m = b != a
        bk0 = (2 * i) % _BANKS
        bk1 = (2 * i + 1) % _BANKS
        plsc.addupdate_scatter(cnt_bs[bk0], [a], ones)
        plsc.addupdate_scatter(s_bs[bk0], [a], lc)
        plsc.addupdate_scatter(cnt_bs[bk1], [b], ones, mask=m)
        plsc.addupdate_scatter(s_bs[bk1], [b], lc, mask=m)
        loss_accs[i % 4] = loss_accs[i % 4] + lo

    cnt_p = zeros
    s_p = zeros
    for bnk in range(_BANKS):
        cnt_p = cnt_p + cnt_bs[bnk][...]
        s_p = s_p + s_bs[bnk][...]
    loss_acc = (loss_accs[0] + loss_accs[1]) + (loss_accs[2] + loss_accs[3])

    part_v[pl.ds(0, _L)] = cnt_p
    part_v[pl.ds(_L, _L)] = s_p
    part_v[pl.ds(2 * _L, _L)] = loss_acc
    pltpu.sync_copy(part_v, shared.at[pl.ds(sid * 3 * _L, 3 * _L)])
    plsc.subcore_barrier()

    @pl.when(sid == 0)
    def _finalize():
        pltpu.sync_copy(shared, all_v)
        pltpu.make_async_copy(w0li_h.at[pl.ds(_TOKENS, _L)], li_v,
                              sem2).wait()
        pltpu.make_async_copy(
            cb_h.at[pl.ds(0, _NUM_LAYERS * _NUM_EXPERTS)],
            caps24_v, sem2).wait()
        pltpu.make_async_copy(
            cb_h.at[pl.ds(_NUM_LAYERS * _NUM_EXPERTS, _NUM_LAYERS)],
            base24_v, sem2).wait()
        li_vec = li_v[...]
        cnt = all_v[pl.ds(0, _L)]
        s = all_v[pl.ds(_L, _L)]
        lsum = all_v[pl.ds(2 * _L, _L)]
        for i in range(1, _NS):
            cnt = cnt + all_v[pl.ds(i * 3 * _L, _L)]
            s = s + all_v[pl.ds((i * 3 + 1) * _L, _L)]
            lsum = lsum + all_v[pl.ds((i * 3 + 2) * _L, _L)]
        avg_loss = jnp.sum(lsum) * (1.0 / _TOKENS)
        base_splat = plsc.load_gather(base24_v, [li_vec])
        new_base = 0.99 * base_splat + 0.01 * avg_loss
        row_idx = li_vec * _NUM_EXPERTS + lanes
        crow = plsc.load_gather(caps24_v, [row_idx])
        caps = crow + new_base * cnt - s
        thr = jnp.sum(caps) * (_TAX_THRESHOLD / _NUM_EXPERTS)
        caps = jnp.where(caps > thr, caps - (caps - thr) * _TAX_RATE, caps)
        caps = jnp.maximum(caps, _MIN_CAP)
        total = jnp.sum(caps)
        scale = jnp.where(total > _TOTAL_CAPITAL * 1.5, 0.95, 1.0)
        shift = jnp.where(total < _TOTAL_CAPITAL * 0.5,
                          _TOTAL_CAPITAL * 0.01, 0.0)
        caps = caps * scale + shift
        plsc.store_scatter(caps24_v, [row_idx], caps)
        plsc.store_scatter(base24_v, [li_vec], new_base, mask=lanes == 0)
        do1 = pltpu.async_copy(caps24_v, caps_out, sem2)
        do2 = pltpu.async_copy(base24_v, base_out, sem2)
        do1.wait()
        do2.wait()


_mesh = plsc.VectorSubcoreMesh(
    core_axis_name="c", subcore_axis_name="s", num_cores=1, num_subcores=_NS)

_sc_call = pl.kernel(
    _sc_body,
    out_type=(
        jax.ShapeDtypeStruct((_NUM_LAYERS * _NUM_EXPERTS,), jnp.float32),
        jax.ShapeDtypeStruct((_NUM_LAYERS,), jnp.float32),
    ),
    mesh=_mesh,
    scratch_types=[
        pltpu.VMEM((_TOK_PER_TILE,), jnp.float32),
        pltpu.VMEM((_TOK_PER_TILE,), jnp.float32),
        pltpu.VMEM((_TOK_PER_TILE,), jnp.int32),
        pltpu.VMEM((_TOK_PER_TILE,), jnp.float32),
        *([pltpu.VMEM((_L,), jnp.float32)] * (2 * _BANKS)),
        pltpu.VMEM((3 * _L,), jnp.float32),
        pltpu.VMEM((_L,), jnp.int32),
        pltpu.VMEM((_NUM_LAYERS,), jnp.float32),
        pltpu.VMEM((_NUM_LAYERS * _NUM_EXPERTS,), jnp.float32),
        pltpu.VMEM_SHARED((_NS * 3 * _L,), jnp.float32),
        pltpu.VMEM((_NS * 3 * _L,), jnp.float32),
        pltpu.SemaphoreType.DMA,
        pltpu.SemaphoreType.DMA,
    ],
    compiler_params=pltpu.CompilerParams(needs_layout_passes=False),
    name="capital_manager_sc",
)


def kernel(capitals, baseline_losses, token_losses, costs, winners, layer_idx):
    w = winners.reshape(_TOKENS, 2)
    ll = jnp.concatenate([
        token_losses.reshape(_TOKENS),
        (token_losses + costs).reshape(_TOKENS),
        lax.bitcast_convert_type(w[:, 1], jnp.float32),
    ])
    w0li = jnp.concatenate([
        w[:, 0],
        jnp.full((_L,), layer_idx, dtype=jnp.int32),
    ])
    cb = jnp.concatenate([
        capitals.reshape(_NUM_LAYERS * _NUM_EXPERTS),
        baseline_losses,
    ])
    caps_flat, new_baselines = _sc_call(ll, w0li, cb)
    return caps_flat.reshape(_NUM_LAYERS, _NUM_EXPERTS), new_baselines
